# baseline (speedup 1.0000x reference)
import jax
import jax.numpy as jnp
from jax.experimental import pallas as pl
from jax.experimental.pallas import tpu as pltpu

B = 8
N = 2048
H = N // 2


def _body(s_ref, m0_ref, m1_ref, out_ref, sums_ref, cnts_ref):
    b = pl.program_id(0)

    s = s_ref[0]
    m0 = m0_ref[0] != 0
    m1 = m1_ref[0] != 0
    part_sum = jnp.sum(jnp.where(m0, s[:H], 0.0)) + jnp.sum(
        jnp.where(m1, s[H:], 0.0)
    )
    part_cnt = jnp.sum(m0.astype(jnp.float32)) + jnp.sum(
        m1.astype(jnp.float32)
    )

    sums_ref[b] = part_sum
    cnts_ref[b] = part_cnt

    @pl.when(b == B - 1)
    def _fin():
        acc = 0.0
        for bb in range(B):
            acc += sums_ref[bb] / cnts_ref[bb]
        out_ref[0, 0] = -acc / B


def kernel(scores, assigns):
    masks = assigns.view(jnp.int8)
    out = pl.pallas_call(
        _body,
        grid=(B,),
        in_specs=[
            pl.BlockSpec((1, N, N), lambda b: (b, 0, 0)),
            pl.BlockSpec((1, H, N), lambda b: (b, 0, 0)),
            pl.BlockSpec((1, H, N), lambda b: (b, 1, 0)),
        ],
        out_specs=pl.BlockSpec(
            (1, 1), lambda b: (0, 0), memory_space=pltpu.SMEM
        ),
        out_shape=jax.ShapeDtypeStruct((1, 1), jnp.float32),
        scratch_shapes=[
            pltpu.SMEM((B,), jnp.float32),
            pltpu.SMEM((B,), jnp.float32),
        ],
    )(scores, masks, masks)
    return out[0, 0]
